# baseline (speedup 1.0000x reference)
import dataclasses
import functools

import jax
import jax.numpy as jnp
from jax import lax
from jax.experimental import pallas as pl
from jax.experimental.pallas import tpu as pltpu
from jax.experimental.pallas import tpu_sc as plsc

BATCH = 16384
EMBED_DIM = 64
NUM_CORES = 2
NUM_SUBCORES = 16
NUM_WORKERS = NUM_CORES * NUM_SUBCORES
ROWS_PER_WORKER = BATCH // NUM_WORKERS
CHUNK = 128
NUM_CHUNKS = ROWS_PER_WORKER // CHUNK
LANES = 16
DBLK = EMBED_DIM // LANES


def _gmf_kernel(user_hbm, item_hbm, uw_hbm, iw_hbm, w_hbm, b_hbm, out_hbm,
                idx_u, idx_i, rows_u, rows_i, wv, bv, out_v, sem):
    wid = lax.axis_index("s") * NUM_CORES + lax.axis_index("c")
    base = wid * ROWS_PER_WORKER

    pltpu.sync_copy(user_hbm.at[wid], idx_u)
    pltpu.sync_copy(item_hbm.at[wid], idx_i)
    pltpu.sync_copy(w_hbm, wv)
    pltpu.sync_copy(b_hbm, bv)

    copies = []
    for j in range(NUM_CHUNKS):
        copies.append(pltpu.async_copy(
            uw_hbm.at[idx_u.at[j]], rows_u.at[pl.ds(j * CHUNK, CHUNK)], sem))
        copies.append(pltpu.async_copy(
            iw_hbm.at[idx_i.at[j]], rows_i.at[pl.ds(j * CHUNK, CHUNK)], sem))
    for c in copies:
        c.wait()

    w_regs = [wv[pl.ds(d * LANES, LANES)] for d in range(DBLK)]
    b_vec = bv[...]
    lane = lax.iota(jnp.int32, LANES)

    @pl.loop(0, ROWS_PER_WORKER // LANES)
    def _(g):
        res = b_vec
        for k in range(LANES):
            r = g * LANES + k
            acc = (rows_u[r, pl.ds(0, LANES)] * rows_i[r, pl.ds(0, LANES)]
                   * w_regs[0])
            for d in range(1, DBLK):
                sl = pl.ds(d * LANES, LANES)
                acc = acc + rows_u[r, sl] * rows_i[r, sl] * w_regs[d]
            res = jnp.where(lane == k, res + jnp.sum(acc), res)
        out_v[pl.ds(g * LANES, LANES)] = res

    pltpu.sync_copy(out_v, out_hbm.at[pl.ds(base, ROWS_PER_WORKER)])


@jax.jit
def kernel(user, item, embed_user_w, embed_item_w, W, b):
    user_r = user.astype(jnp.int32).reshape(NUM_WORKERS, NUM_CHUNKS, CHUNK)
    item_r = item.astype(jnp.int32).reshape(NUM_WORKERS, NUM_CHUNKS, CHUNK)
    w_flat = W.reshape(EMBED_DIM)
    b_pad = jnp.broadcast_to(b, (LANES,))

    mesh = plsc.VectorSubcoreMesh(core_axis_name="c", subcore_axis_name="s")
    cp = pltpu.CompilerParams()
    if "needs_layout_passes" in pltpu.CompilerParams.__dataclass_fields__:
        cp = dataclasses.replace(cp, needs_layout_passes=False)
    if "use_tc_tiling_on_sc" in pltpu.CompilerParams.__dataclass_fields__:
        cp = dataclasses.replace(cp, use_tc_tiling_on_sc=False)
    run = pl.kernel(
        _gmf_kernel,
        out_type=jax.ShapeDtypeStruct((BATCH,), jnp.float32),
        mesh=mesh,
        compiler_params=cp,
        scratch_types=[
            pltpu.VMEM((NUM_CHUNKS, CHUNK), jnp.int32),
            pltpu.VMEM((NUM_CHUNKS, CHUNK), jnp.int32),
            pltpu.VMEM((ROWS_PER_WORKER, EMBED_DIM), jnp.float32),
            pltpu.VMEM((ROWS_PER_WORKER, EMBED_DIM), jnp.float32),
            pltpu.VMEM((EMBED_DIM,), jnp.float32),
            pltpu.VMEM((LANES,), jnp.float32),
            pltpu.VMEM((ROWS_PER_WORKER,), jnp.float32),
            pltpu.SemaphoreType.DMA,
        ],
    )
    return run(user_r, item_r, embed_user_w, embed_item_w, w_flat, b_pad)
